# baseline (speedup 1.0000x reference)
import functools

import jax
import jax.numpy as jnp
from jax import lax
from jax.experimental import pallas as pl
from jax.experimental.pallas import tpu as pltpu
from jax.experimental.pallas import tpu_sc as plsc

_L = 16
_C = 80


def _sc_body(epw, num_cores, h_src, h_dst, s_idx, d_idx, out,
             sidx_v, didx_v, srows, drows, mat_v, scores_v, sems, semd):
    wid = lax.axis_index("s") * num_cores + lax.axis_index("c")
    nchk = epw // _C
    wbase = wid * epw

    pltpu.sync_copy(s_idx.at[pl.ds(wbase, epw)], sidx_v)
    pltpu.sync_copy(d_idx.at[pl.ds(wbase, epw)], didx_v)

    def start(chunk, b):
        pltpu.async_copy(h_src.at[sidx_v.at[pl.ds(chunk * _C, _C)]],
                         srows[b], sems[b])
        pltpu.async_copy(h_dst.at[didx_v.at[pl.ds(chunk * _C, _C)]],
                         drows[b], semd[b])

    start(0, 0)
    start(1, 1)

    lanes = lax.iota(jnp.int32, _L)

    def compute(chunk, b):
        sr, dr = srows[b], drows[b]

        def group_body(g, _):
            accs = []
            for e16 in range(_L):
                e = g * _L + e16
                a0 = a1 = None
                for k in range(4):
                    sw = plsc.bitcast(sr[e, pl.ds(k * _L, _L)], jnp.bfloat16)
                    dw = plsc.bitcast(dr[e, pl.ds(k * _L, _L)], jnp.bfloat16)
                    pa, pb = plsc.unpack(sw * dw,
                                         format=plsc.PackFormat.INTERLEAVED)
                    a0 = pa if a0 is None else a0 + pa
                    a1 = pb if a1 is None else a1 + pb
                accs.append(a0 + a1)
            for e16 in range(_L):
                mat_v[pl.ds(e16 * _L, _L)] = accs[e16]
            tot = plsc.load_gather(mat_v, [lanes * _L])
            for j in range(1, _L):
                tot = tot + plsc.load_gather(mat_v, [lanes * _L + j])
            scores_v[pl.ds(chunk * _C + g * _L, _L)] = tot
            return 0

        lax.fori_loop(0, _C // _L, group_body, 0)

    def pair_body(i2, _):
        for b in range(2):
            i = i2 * 2 + b

            @pl.when(i < nchk)
            def _():
                pltpu.make_async_copy(
                    h_src.at[sidx_v.at[pl.ds(i * _C, _C)]], srows[b], sems[b]
                ).wait()
                pltpu.make_async_copy(
                    h_dst.at[didx_v.at[pl.ds(i * _C, _C)]], drows[b], semd[b]
                ).wait()
                compute(i, b)

                @pl.when(i + 2 < nchk)
                def _():
                    start(i + 2, b)
        return 0

    lax.fori_loop(0, (nchk + 1) // 2, pair_body, 0)
    pltpu.sync_copy(scores_v, out.at[pl.ds(wbase, epw)])


def kernel(h_src, h_dst, edge_index):
    n_nodes, d_feat = h_src.shape
    n_edges = edge_index.shape[1]
    assert d_feat == 128

    s_idx = edge_index[0].astype(jnp.int32)
    d_idx = edge_index[1].astype(jnp.int32)
    def _pack_bf16_pairs(h):
        u = lax.bitcast_convert_type(h, jnp.uint32)
        r = (u + 0x7FFF + ((u >> 16) & 1)) >> 16
        lo, hi = r[:, 0::2], r[:, 1::2]
        return lax.bitcast_convert_type(lo | (hi << 16), jnp.int32)

    h_src = _pack_bf16_pairs(h_src)
    h_dst = _pack_bf16_pairs(h_dst)

    mesh = plsc.VectorSubcoreMesh(core_axis_name="c", subcore_axis_name="s")
    num_cores = mesh.num_cores
    nw = num_cores * mesh.num_subcores
    assert n_edges % (nw * _C) == 0
    epw = n_edges // nw

    sc_fn = pl.kernel(
        functools.partial(_sc_body, epw, num_cores),
        out_type=jax.ShapeDtypeStruct((n_edges,), jnp.float32),
        mesh=mesh,
        scratch_types=[
            pltpu.VMEM((epw,), jnp.int32),
            pltpu.VMEM((epw,), jnp.int32),
            [pltpu.VMEM((_C, 64), jnp.int32) for _ in range(2)],
            [pltpu.VMEM((_C, 64), jnp.int32) for _ in range(2)],
            pltpu.VMEM((_L * _L,), jnp.float32),
            pltpu.VMEM((epw,), jnp.float32),
            [pltpu.SemaphoreType.DMA for _ in range(2)],
            [pltpu.SemaphoreType.DMA for _ in range(2)],
        ],
        compiler_params=pltpu.CompilerParams(
            needs_layout_passes=False, use_tc_tiling_on_sc=False),
    )
    scores = sc_fn(h_src, h_dst, s_idx, d_idx)
    return scores.reshape(n_edges, 1)


# speedup vs baseline: 3.5351x; 3.5351x over previous
import functools

import jax
import jax.numpy as jnp
from jax import lax
from jax.experimental import pallas as pl
from jax.experimental.pallas import tpu as pltpu
from jax.experimental.pallas import tpu_sc as plsc

_L = 16
_C = 80


def _sc_body(epw, num_cores, h_src, h_dst, s_idx, d_idx, out,
             sidx_v, didx_v, srows, drows, mat_v, scores_v, sems, semd):
    wid = lax.axis_index("s") * num_cores + lax.axis_index("c")
    nchk = epw // _C
    wbase = wid * epw

    pltpu.sync_copy(s_idx.at[pl.ds(wbase, epw)], sidx_v)
    pltpu.sync_copy(d_idx.at[pl.ds(wbase, epw)], didx_v)

    def start(chunk, b):
        pltpu.async_copy(h_src.at[sidx_v.at[pl.ds(chunk * _C, _C)]],
                         srows[b], sems[b])
        pltpu.async_copy(h_dst.at[didx_v.at[pl.ds(chunk * _C, _C)]],
                         drows[b], semd[b])

    start(0, 0)
    start(1, 1)

    lanes = lax.iota(jnp.int32, _L)

    def compute(chunk, b):
        sr, dr = srows[b], drows[b]

        def group_body(g, _):
            accs = []
            for e16 in range(_L):
                e = g * _L + e16
                a0 = a1 = None
                for k in range(4):
                    sw = plsc.bitcast(sr[e, pl.ds(k * _L, _L)], jnp.bfloat16)
                    dw = plsc.bitcast(dr[e, pl.ds(k * _L, _L)], jnp.bfloat16)
                    pa, pb = plsc.unpack(sw * dw,
                                         format=plsc.PackFormat.INTERLEAVED)
                    a0 = pa if a0 is None else a0 + pa
                    a1 = pb if a1 is None else a1 + pb
                accs.append(a0 + a1)
            for e16 in range(_L):
                mat_v[pl.ds(e16 * _L, _L)] = accs[e16]
            tot = plsc.load_gather(mat_v, [lanes * _L])
            for j in range(1, _L):
                tot = tot + plsc.load_gather(mat_v, [lanes * _L + j])
            scores_v[pl.ds(chunk * _C + g * _L, _L)] = tot
            return 0

        lax.fori_loop(0, _C // _L, group_body, 0)

    def pair_body(i2, _):
        for b in range(2):
            i = i2 * 2 + b

            @pl.when(i < nchk)
            def _():
                pltpu.make_async_copy(
                    h_src.at[sidx_v.at[pl.ds(i * _C, _C)]], srows[b], sems[b]
                ).wait()
                pltpu.make_async_copy(
                    h_dst.at[didx_v.at[pl.ds(i * _C, _C)]], drows[b], semd[b]
                ).wait()
                compute(i, b)

                @pl.when(i + 2 < nchk)
                def _():
                    start(i + 2, b)
        return 0

    lax.fori_loop(0, (nchk + 1) // 2, pair_body, 0)
    pltpu.sync_copy(scores_v, out.at[pl.ds(wbase, epw)])


def kernel(h_src, h_dst, edge_index):
    n_nodes, d_feat = h_src.shape
    n_edges = edge_index.shape[1]
    assert d_feat == 128

    s_idx = edge_index[0].astype(jnp.int32)
    d_idx = edge_index[1].astype(jnp.int32)
    def _pack_bf16_pairs(h):
        u = lax.bitcast_convert_type(h, jnp.uint32)
        r = (u + 0x7FFF + ((u >> 16) & 1)) >> 16
        lo, hi = r[:, :64], r[:, 64:]
        return lax.bitcast_convert_type(lo | (hi << 16), jnp.int32)

    h_src = _pack_bf16_pairs(h_src)
    h_dst = _pack_bf16_pairs(h_dst)

    mesh = plsc.VectorSubcoreMesh(core_axis_name="c", subcore_axis_name="s")
    num_cores = mesh.num_cores
    nw = num_cores * mesh.num_subcores
    assert n_edges % (nw * _C) == 0
    epw = n_edges // nw

    sc_fn = pl.kernel(
        functools.partial(_sc_body, epw, num_cores),
        out_type=jax.ShapeDtypeStruct((n_edges,), jnp.float32),
        mesh=mesh,
        scratch_types=[
            pltpu.VMEM((epw,), jnp.int32),
            pltpu.VMEM((epw,), jnp.int32),
            [pltpu.VMEM((_C, 64), jnp.int32) for _ in range(2)],
            [pltpu.VMEM((_C, 64), jnp.int32) for _ in range(2)],
            pltpu.VMEM((_L * _L,), jnp.float32),
            pltpu.VMEM((epw,), jnp.float32),
            [pltpu.SemaphoreType.DMA for _ in range(2)],
            [pltpu.SemaphoreType.DMA for _ in range(2)],
        ],
        compiler_params=pltpu.CompilerParams(
            needs_layout_passes=False, use_tc_tiling_on_sc=False),
    )
    scores = sc_fn(h_src, h_dst, s_idx, d_idx)
    return scores.reshape(n_edges, 1)


# speedup vs baseline: 3.7754x; 1.0680x over previous
import functools

import jax
import jax.numpy as jnp
from jax import lax
from jax.experimental import pallas as pl
from jax.experimental.pallas import tpu as pltpu
from jax.experimental.pallas import tpu_sc as plsc

_L = 16
_C = 80


def _sc_body(epw, num_cores, h_src, h_dst, e_idx, out,
             sidx_v, didx_v, srows, drows, mat_v, scores_v, sems, semd):
    wid = lax.axis_index("s") * num_cores + lax.axis_index("c")
    nchk = epw // _C
    wbase = wid * epw

    pltpu.sync_copy(e_idx.at[0, pl.ds(wbase, epw)], sidx_v)
    pltpu.sync_copy(e_idx.at[1, pl.ds(wbase, epw)], didx_v)

    def start(chunk, b):
        pltpu.async_copy(h_src.at[sidx_v.at[pl.ds(chunk * _C, _C)]],
                         srows[b], sems[b])
        pltpu.async_copy(h_dst.at[didx_v.at[pl.ds(chunk * _C, _C)]],
                         drows[b], semd[b])

    start(0, 0)
    start(1, 1)

    lanes = lax.iota(jnp.int32, _L)

    def compute(chunk, b):
        sr, dr = srows[b], drows[b]

        def group_body(g, _):
            accs = []
            for e16 in range(_L):
                e = g * _L + e16
                a0 = a1 = None
                for k in range(4):
                    sw = plsc.bitcast(sr[e, pl.ds(k * _L, _L)], jnp.bfloat16)
                    dw = plsc.bitcast(dr[e, pl.ds(k * _L, _L)], jnp.bfloat16)
                    pa, pb = plsc.unpack(sw * dw,
                                         format=plsc.PackFormat.INTERLEAVED)
                    a0 = pa if a0 is None else a0 + pa
                    a1 = pb if a1 is None else a1 + pb
                accs.append(a0 + a1)
            for e16 in range(_L):
                mat_v[pl.ds(e16 * _L, _L)] = accs[e16]
            tot = plsc.load_gather(mat_v, [lanes * _L])
            for j in range(1, _L):
                tot = tot + plsc.load_gather(mat_v, [lanes * _L + j])
            scores_v[pl.ds(chunk * _C + g * _L, _L)] = tot
            return 0

        lax.fori_loop(0, _C // _L, group_body, 0)

    def pair_body(i2, _):
        for b in range(2):
            i = i2 * 2 + b

            @pl.when(i < nchk)
            def _():
                pltpu.make_async_copy(
                    h_src.at[sidx_v.at[pl.ds(i * _C, _C)]], srows[b], sems[b]
                ).wait()
                pltpu.make_async_copy(
                    h_dst.at[didx_v.at[pl.ds(i * _C, _C)]], drows[b], semd[b]
                ).wait()
                compute(i, b)

                @pl.when(i + 2 < nchk)
                def _():
                    start(i + 2, b)
        return 0

    lax.fori_loop(0, (nchk + 1) // 2, pair_body, 0)
    pltpu.sync_copy(scores_v, out.at[pl.ds(wbase, epw)])


def kernel(h_src, h_dst, edge_index):
    n_nodes, d_feat = h_src.shape
    n_edges = edge_index.shape[1]
    assert d_feat == 128

    e_idx = edge_index.astype(jnp.int32)
    def _pack_bf16_pairs(h):
        u = lax.bitcast_convert_type(h, jnp.uint32)
        r = (u + 0x7FFF + ((u >> 16) & 1)) >> 16
        lo, hi = r[:, :64], r[:, 64:]
        return lax.bitcast_convert_type(lo | (hi << 16), jnp.int32)

    h_src = _pack_bf16_pairs(h_src)
    h_dst = _pack_bf16_pairs(h_dst)

    mesh = plsc.VectorSubcoreMesh(core_axis_name="c", subcore_axis_name="s")
    num_cores = mesh.num_cores
    nw = num_cores * mesh.num_subcores
    assert n_edges % (nw * _C) == 0
    epw = n_edges // nw

    sc_fn = pl.kernel(
        functools.partial(_sc_body, epw, num_cores),
        out_type=jax.ShapeDtypeStruct((n_edges,), jnp.float32),
        mesh=mesh,
        scratch_types=[
            pltpu.VMEM((epw,), jnp.int32),
            pltpu.VMEM((epw,), jnp.int32),
            [pltpu.VMEM((_C, 64), jnp.int32) for _ in range(2)],
            [pltpu.VMEM((_C, 64), jnp.int32) for _ in range(2)],
            pltpu.VMEM((_L * _L,), jnp.float32),
            pltpu.VMEM((epw,), jnp.float32),
            [pltpu.SemaphoreType.DMA for _ in range(2)],
            [pltpu.SemaphoreType.DMA for _ in range(2)],
        ],
        compiler_params=pltpu.CompilerParams(
            needs_layout_passes=False, use_tc_tiling_on_sc=False),
    )
    return sc_fn(h_src, h_dst, e_idx).reshape(n_edges, 1)


# speedup vs baseline: 4.2792x; 1.1335x over previous
import functools

import jax
import jax.numpy as jnp
from jax import lax
from jax.experimental import pallas as pl
from jax.experimental.pallas import tpu as pltpu
from jax.experimental.pallas import tpu_sc as plsc

_L = 16
_C = 400


def _sc_body(epw, num_cores, h_src, h_dst, e_idx, out,
             sidx_v, didx_v, srows, drows, mat_v, sbufs, sems, semd, semo):
    wid = lax.axis_index("s") * num_cores + lax.axis_index("c")
    nchk = epw // _C
    wbase = wid * epw

    pltpu.sync_copy(e_idx.at[0, pl.ds(wbase, epw)], sidx_v)
    pltpu.sync_copy(e_idx.at[1, pl.ds(wbase, epw)], didx_v)

    def start(chunk, b):
        pltpu.async_copy(h_src.at[sidx_v.at[pl.ds(chunk * _C, _C)]],
                         srows[b], sems[b])
        pltpu.async_copy(h_dst.at[didx_v.at[pl.ds(chunk * _C, _C)]],
                         drows[b], semd[b])

    start(0, 0)
    start(1, 1)

    lanes = lax.iota(jnp.int32, _L)

    def compute(chunk, b):
        sr, dr = srows[b], drows[b]
        sb = sbufs[b]

        def group_body(g, _):
            accs = []
            for e16 in range(_L):
                e = g * _L + e16
                a0 = a1 = None
                for k in range(4):
                    sw = plsc.bitcast(sr[e, pl.ds(k * _L, _L)], jnp.bfloat16)
                    dw = plsc.bitcast(dr[e, pl.ds(k * _L, _L)], jnp.bfloat16)
                    pa, pb = plsc.unpack(sw * dw,
                                         format=plsc.PackFormat.INTERLEAVED)
                    a0 = pa if a0 is None else a0 + pa
                    a1 = pb if a1 is None else a1 + pb
                accs.append(a0 + a1)
            for e16 in range(_L):
                mat_v[pl.ds(e16 * _L, _L)] = accs[e16]
            tot = plsc.load_gather(mat_v, [lanes * _L])
            for j in range(1, _L):
                tot = tot + plsc.load_gather(mat_v, [lanes * _L + j])
            sb[pl.ds(g * _L, _L)] = tot
            return 0

        lax.fori_loop(0, _C // _L, group_body, 0)

    def pair_body(i2, _):
        for b in range(2):
            i = i2 * 2 + b

            @pl.when(i < nchk)
            def _():
                pltpu.make_async_copy(
                    h_src.at[sidx_v.at[pl.ds(i * _C, _C)]], srows[b], sems[b]
                ).wait()
                pltpu.make_async_copy(
                    h_dst.at[didx_v.at[pl.ds(i * _C, _C)]], drows[b], semd[b]
                ).wait()

                @pl.when(i >= 2)
                def _():
                    pltpu.make_async_copy(
                        sbufs[b], out.at[pl.ds(wbase + (i - 2) * _C, _C)],
                        semo[b]).wait()

                compute(i, b)
                pltpu.async_copy(
                    sbufs[b], out.at[pl.ds(wbase + i * _C, _C)], semo[b])

                @pl.when(i + 2 < nchk)
                def _():
                    start(i + 2, b)
        return 0

    lax.fori_loop(0, (nchk + 1) // 2, pair_body, 0)
    for b in range(2):
        cb = nchk - 1 if (nchk - 1) % 2 == b else nchk - 2
        if cb >= 0:
            pltpu.make_async_copy(
                sbufs[b], out.at[pl.ds(wbase + cb * _C, _C)], semo[b]).wait()


def kernel(h_src, h_dst, edge_index):
    n_nodes, d_feat = h_src.shape
    n_edges = edge_index.shape[1]
    assert d_feat == 128

    e_idx = edge_index.astype(jnp.int32)
    def _pack_bf16_pairs(h):
        u = lax.bitcast_convert_type(h, jnp.uint32)
        r = (u + 0x7FFF + ((u >> 16) & 1)) >> 16
        lo, hi = r[:, :64], r[:, 64:]
        return lax.bitcast_convert_type(lo | (hi << 16), jnp.int32)

    h_src = _pack_bf16_pairs(h_src)
    h_dst = _pack_bf16_pairs(h_dst)

    mesh = plsc.VectorSubcoreMesh(core_axis_name="c", subcore_axis_name="s")
    num_cores = mesh.num_cores
    nw = num_cores * mesh.num_subcores
    assert n_edges % (nw * _C) == 0
    epw = n_edges // nw

    sc_fn = pl.kernel(
        functools.partial(_sc_body, epw, num_cores),
        out_type=jax.ShapeDtypeStruct((n_edges,), jnp.float32),
        mesh=mesh,
        scratch_types=[
            pltpu.VMEM((epw,), jnp.int32),
            pltpu.VMEM((epw,), jnp.int32),
            [pltpu.VMEM((_C, 64), jnp.int32) for _ in range(2)],
            [pltpu.VMEM((_C, 64), jnp.int32) for _ in range(2)],
            pltpu.VMEM((_L * _L,), jnp.float32),
            [pltpu.VMEM((_C,), jnp.float32) for _ in range(2)],
            [pltpu.SemaphoreType.DMA for _ in range(2)],
            [pltpu.SemaphoreType.DMA for _ in range(2)],
            [pltpu.SemaphoreType.DMA for _ in range(2)],
        ],
        compiler_params=pltpu.CompilerParams(
            needs_layout_passes=False, use_tc_tiling_on_sc=False),
    )
    return sc_fn(h_src, h_dst, e_idx).reshape(n_edges, 1)


# speedup vs baseline: 4.2793x; 1.0000x over previous
import functools

import jax
import jax.numpy as jnp
from jax import lax
from jax.experimental import pallas as pl
from jax.experimental.pallas import tpu as pltpu
from jax.experimental.pallas import tpu_sc as plsc

_L = 16
_C = 400


def _sc_body(epw, num_cores, h_src, h_dst, e_idx, out,
             sidx_v, didx_v, srows, drows, mat_v, sbufs, sems, semd, semo):
    wid = lax.axis_index("s") * num_cores + lax.axis_index("c")
    nchk = epw // _C
    wbase = wid * epw

    pltpu.sync_copy(e_idx.at[0, pl.ds(wbase, epw)], sidx_v)
    pltpu.sync_copy(e_idx.at[1, pl.ds(wbase, epw)], didx_v)

    def start(chunk, b):
        pltpu.async_copy(h_src.at[sidx_v.at[pl.ds(chunk * _C, _C)]],
                         srows[b], sems[b])
        pltpu.async_copy(h_dst.at[didx_v.at[pl.ds(chunk * _C, _C)]],
                         drows[b], semd[b])

    start(0, 0)
    start(1, 1)

    lanes = lax.iota(jnp.int32, _L)

    def compute(chunk, b):
        sr, dr = srows[b], drows[b]
        sb = sbufs[b]

        def group_body(g, _):
            accs = []
            for e16 in range(_L):
                e = g * _L + e16
                a0 = a1 = None
                for k in range(4):
                    sw = plsc.bitcast(sr[e, pl.ds(k * _L, _L)], jnp.bfloat16)
                    dw = plsc.bitcast(dr[e, pl.ds(k * _L, _L)], jnp.bfloat16)
                    pa, pb = plsc.unpack(sw * dw,
                                         format=plsc.PackFormat.INTERLEAVED)
                    a0 = pa if a0 is None else a0 + pa
                    a1 = pb if a1 is None else a1 + pb
                accs.append(a0 + a1)
            for e16 in range(_L):
                mat_v[pl.ds(e16 * _L, _L)] = accs[e16]
            tot = plsc.load_gather(mat_v, [lanes * _L])
            for j in range(1, _L):
                tot = tot + plsc.load_gather(mat_v, [lanes * _L + j])
            sb[pl.ds(g * _L, _L)] = tot
            return 0

        lax.fori_loop(0, _C // _L, group_body, 0)

    def pair_body(i2, _):
        for b in range(2):
            i = i2 * 2 + b

            @pl.when(i < nchk)
            def _():
                pltpu.make_async_copy(
                    h_src.at[sidx_v.at[pl.ds(i * _C, _C)]], srows[b], sems[b]
                ).wait()
                pltpu.make_async_copy(
                    h_dst.at[didx_v.at[pl.ds(i * _C, _C)]], drows[b], semd[b]
                ).wait()

                @pl.when(i >= 2)
                def _():
                    pltpu.make_async_copy(
                        sbufs[b], out.at[0, pl.ds(wbase + (i - 2) * _C, _C)],
                        semo[b]).wait()

                compute(i, b)
                pltpu.async_copy(
                    sbufs[b], out.at[0, pl.ds(wbase + i * _C, _C)], semo[b])

                @pl.when(i + 2 < nchk)
                def _():
                    start(i + 2, b)
        return 0

    lax.fori_loop(0, (nchk + 1) // 2, pair_body, 0)
    for b in range(2):
        cb = nchk - 1 if (nchk - 1) % 2 == b else nchk - 2
        if cb >= 0:
            pltpu.make_async_copy(
                sbufs[b], out.at[0, pl.ds(wbase + cb * _C, _C)], semo[b]).wait()


def kernel(h_src, h_dst, edge_index):
    n_nodes, d_feat = h_src.shape
    n_edges = edge_index.shape[1]
    assert d_feat == 128

    e_idx = edge_index.astype(jnp.int32)
    def _pack_bf16_pairs(h):
        u = lax.bitcast_convert_type(h, jnp.uint32)
        r = (u + 0x7FFF + ((u >> 16) & 1)) >> 16
        lo, hi = r[:, :64], r[:, 64:]
        return lax.bitcast_convert_type(lo | (hi << 16), jnp.int32)

    h_src = _pack_bf16_pairs(h_src)
    h_dst = _pack_bf16_pairs(h_dst)

    mesh = plsc.VectorSubcoreMesh(core_axis_name="c", subcore_axis_name="s")
    num_cores = mesh.num_cores
    nw = num_cores * mesh.num_subcores
    assert n_edges % (nw * _C) == 0
    epw = n_edges // nw

    sc_fn = pl.kernel(
        functools.partial(_sc_body, epw, num_cores),
        out_type=jax.ShapeDtypeStruct((1, n_edges), jnp.float32),
        mesh=mesh,
        scratch_types=[
            pltpu.VMEM((epw,), jnp.int32),
            pltpu.VMEM((epw,), jnp.int32),
            [pltpu.VMEM((_C, 64), jnp.int32) for _ in range(2)],
            [pltpu.VMEM((_C, 64), jnp.int32) for _ in range(2)],
            pltpu.VMEM((_L * _L,), jnp.float32),
            [pltpu.VMEM((_C,), jnp.float32) for _ in range(2)],
            [pltpu.SemaphoreType.DMA for _ in range(2)],
            [pltpu.SemaphoreType.DMA for _ in range(2)],
            [pltpu.SemaphoreType.DMA for _ in range(2)],
        ],
        compiler_params=pltpu.CompilerParams(
            needs_layout_passes=False, use_tc_tiling_on_sc=False),
    )
    return sc_fn(h_src, h_dst, e_idx).reshape(n_edges, 1)
